# Initial kernel scaffold; baseline (speedup 1.0000x reference)
import jax
import jax.numpy as jnp
from jax.experimental import pallas as pl


def kernel(p0, p1, p2):
    raise NotImplementedError("write your pallas kernel here")


# speedup vs baseline: 3.7931x; 3.7931x over previous
import numpy as np
import jax
import jax.numpy as jnp
from jax.experimental import pallas as pl
from jax.experimental.pallas import tpu as pltpu

_NC = 80
_THR = 0.5
_MAXB = 100
_IOU = 0.1
_NEG = -1e30
_ANCH = np.array([[10., 13.], [16., 30.], [33., 23.], [30., 61.], [62., 45.],
                  [59., 119.], [116., 90.], [156., 198.], [373., 326.]],
                 dtype=np.float32)
_N = 3 * (19 * 19 + 38 * 38 + 76 * 76)
_NPAD = 22784
_SLOTS = 128
_OROWS = 104


def _build_consts():
    offx_l, offy_l, sx_l, sy_l, rw_l, rh_l = [], [], [], [], [], []
    for g, base in ((19, 6), (38, 3), (76, 0)):
        ratio = np.array([608.0 / g, 608.0 / g], dtype=np.float32)
        resc = (_ANCH[base:base + 3]
                / np.stack([ratio[1], ratio[0]]).astype(np.float32))
        resc = resc.astype(np.float32)
        shp = (g, g, 3)
        offx_l.append(np.broadcast_to(
            np.arange(g, dtype=np.float32)[None, :, None], shp).ravel())
        offy_l.append(np.broadcast_to(
            np.arange(g, dtype=np.float32)[:, None, None], shp).ravel())
        sx_l.append(np.full(shp, ratio[1], np.float32).ravel())
        sy_l.append(np.full(shp, ratio[0], np.float32).ravel())
        rw_l.append(np.broadcast_to(resc[:, 0], shp).ravel())
        rh_l.append(np.broadcast_to(resc[:, 1], shp).ravel())
    rows = []
    for lst in (offx_l, offy_l, sx_l, sy_l, rw_l, rh_l):
        v = np.concatenate(lst).astype(np.float32)
        rows.append(np.pad(v, (0, _NPAD - _N)))
    rows.append(np.zeros(_NPAD, np.float32))
    rows.append(np.zeros(_NPAD, np.float32))
    return np.stack(rows)


_CONSTS = _build_consts()


def _predict_kernel(fm_ref, cst_ref, boxes_out, scores_out, labels_out,
                    work, bxs, selsc, selix):
    offx = cst_ref[0:1, :]
    offy = cst_ref[1:2, :]
    sx = cst_ref[2:3, :]
    sy = cst_ref[3:4, :]
    rw = cst_ref[4:5, :]
    rh = cst_ref[5:6, :]
    tx = fm_ref[0:1, :]
    ty = fm_ref[1:2, :]
    tw = fm_ref[2:3, :]
    th = fm_ref[3:4, :]
    conf = fm_ref[4:5, :]
    cx = (jax.nn.sigmoid(tx) + offx) * sx
    cy = (jax.nn.sigmoid(ty) + offy) * sy
    w = (jnp.exp(tw) * rw) * sx
    h = (jnp.exp(th) * rh) * sy
    x1 = cx - w / 2
    y1 = cy - h / 2
    x2 = cx + w / 2
    y2 = cy + h / 2
    bxs[0:1, :] = x1
    bxs[1:2, :] = y1
    bxs[2:3, :] = x2
    bxs[3:4, :] = y2
    bxs[4:5, :] = (x2 - x1) * (y2 - y1)

    col = jax.lax.broadcasted_iota(jnp.int32, (1, _NPAD), 1)
    sc = jax.nn.sigmoid(fm_ref[5:5 + _NC, :]) * jax.nn.sigmoid(conf)
    work[:, :] = jnp.where((sc > _THR) & (col < _N), sc, _NEG)
    selsc[:, :] = jnp.full((_NC, _SLOTS), _NEG, jnp.float32)
    selix[:, :] = jnp.zeros((_NC, _SLOTS), jnp.float32)

    iota_l = jax.lax.broadcasted_iota(jnp.int32, (_NC, _NPAD), 1)
    slot_l = jax.lax.broadcasted_iota(jnp.int32, (_NC, _SLOTS), 1)

    boxes_out[:, :] = jnp.full((_OROWS, 128), -1.0, jnp.float32)
    scores_out[:, :] = jnp.full((_OROWS, 128), -1.0, jnp.float32)
    labels_out[:, :] = jnp.full((_OROWS, 128), -1.0, jnp.float32)

    def nms_body(carry):
        k, _ = carry
        wk = work[:, :]
        m = jnp.max(wk, axis=1, keepdims=True)
        eq = wk == m
        i = jnp.min(jnp.where(eq, iota_l, 2 ** 30), axis=1, keepdims=True)
        eq2 = iota_l == i
        X1 = bxs[0:1, :]
        Y1 = bxs[1:2, :]
        X2 = bxs[2:3, :]
        Y2 = bxs[3:4, :]
        sel_x1 = jnp.sum(jnp.where(eq2, X1, 0.0), axis=1, keepdims=True)
        sel_y1 = jnp.sum(jnp.where(eq2, Y1, 0.0), axis=1, keepdims=True)
        sel_x2 = jnp.sum(jnp.where(eq2, X2, 0.0), axis=1, keepdims=True)
        sel_y2 = jnp.sum(jnp.where(eq2, Y2, 0.0), axis=1, keepdims=True)
        a1 = (sel_x2 - sel_x1) * (sel_y2 - sel_y1)
        xx1 = jnp.maximum(sel_x1, X1)
        yy1 = jnp.maximum(sel_y1, Y1)
        xx2 = jnp.minimum(sel_x2, X2)
        yy2 = jnp.minimum(sel_y2, Y2)
        inter = jnp.maximum(xx2 - xx1, 0.0) * jnp.maximum(yy2 - yy1, 0.0)
        union = a1 + bxs[4:5, :] - inter
        iou = jnp.where(union > 0, inter / jnp.maximum(union, 1e-9), 0.0)
        work[:, :] = jnp.where((iou > _IOU) | eq2, _NEG, wk)
        onek = slot_l == k
        selsc[:, :] = jnp.where(onek, m, selsc[:, :])
        selix[:, :] = jnp.where(onek, i.astype(jnp.float32), selix[:, :])
        return k + 1, jnp.max(m) > _NEG / 2

    jax.lax.while_loop(lambda c: (c[0] < _MAXB) & c[1], nms_body, (0, True))

    keyc = jax.lax.broadcasted_iota(jnp.int32, (_NC, _SLOTS), 0)
    keys = jax.lax.broadcasted_iota(jnp.int32, (_NC, _SLOTS), 1)
    key = keyc * _SLOTS + keys
    lane = jax.lax.broadcasted_iota(jnp.int32, (1, 128), 1)
    iota_b = jax.lax.broadcasted_iota(jnp.int32, (8, _NPAD), 1)
    rowi = jax.lax.broadcasted_iota(jnp.int32, (_OROWS, 128), 0)

    def topk_body(t, _):
        s2 = selsc[:, :]
        m = jnp.max(s2)
        eq = s2 == m
        pos = jnp.min(jnp.where(eq, key, 2 ** 30))
        hit = eq & (key == pos)
        cls = jnp.sum(jnp.where(hit, keyc, 0)).astype(jnp.float32)
        nidx = jnp.sum(jnp.where(hit, selix[:, :], 0.0))
        n = nidx.astype(jnp.int32)
        selb = jnp.sum(jnp.where(iota_b == n, bxs[:, :], 0.0),
                       axis=1, keepdims=True)
        invalid = m < _NEG / 2
        row = jnp.where(lane == 0, jnp.where(invalid, -1.0, selb[0, 0]),
              jnp.where(lane == 1, jnp.where(invalid, -1.0, selb[1, 0]),
              jnp.where(lane == 2, jnp.where(invalid, -1.0, selb[2, 0]),
                        jnp.where(invalid, -1.0, selb[3, 0]))))
        hit_row = rowi == t
        boxes_out[:, :] = jnp.where(hit_row, row, boxes_out[:, :])
        scores_out[:, :] = jnp.where(
            hit_row, jnp.where(invalid, -1.0, m), scores_out[:, :])
        labels_out[:, :] = jnp.where(
            hit_row, jnp.where(invalid, -1.0, cls), labels_out[:, :])
        selsc[:, :] = jnp.where(hit, _NEG, s2)
        return 0

    jax.lax.fori_loop(0, _MAXB, topk_body, 0)


def kernel(p0, p1, p2):
    B = p0.shape[0]
    fm = jnp.concatenate(
        [p.reshape(B, -1, 5 + _NC) for p in (p0, p1, p2)], axis=1)
    fmt = jnp.transpose(fm, (0, 2, 1))
    fmt = jnp.pad(fmt, ((0, 0), (0, 3), (0, _NPAD - _N)))
    consts = jnp.asarray(_CONSTS)

    bo, so, lo = pl.pallas_call(
        _predict_kernel,
        grid=(B,),
        in_specs=[
            pl.BlockSpec((None, 88, _NPAD), lambda i: (i, 0, 0)),
            pl.BlockSpec((8, _NPAD), lambda i: (0, 0)),
        ],
        out_specs=[
            pl.BlockSpec((None, _OROWS, 128), lambda i: (i, 0, 0)),
            pl.BlockSpec((None, _OROWS, 128), lambda i: (i, 0, 0)),
            pl.BlockSpec((None, _OROWS, 128), lambda i: (i, 0, 0)),
        ],
        out_shape=[
            jax.ShapeDtypeStruct((B, _OROWS, 128), jnp.float32),
            jax.ShapeDtypeStruct((B, _OROWS, 128), jnp.float32),
            jax.ShapeDtypeStruct((B, _OROWS, 128), jnp.float32),
        ],
        scratch_shapes=[
            pltpu.VMEM((_NC, _NPAD), jnp.float32),
            pltpu.VMEM((8, _NPAD), jnp.float32),
            pltpu.VMEM((_NC, _SLOTS), jnp.float32),
            pltpu.VMEM((_NC, _SLOTS), jnp.float32),
        ],
    )(fmt, consts)

    out_boxes = bo[:, :_MAXB, :4]
    out_scores = so[:, :_MAXB, 0]
    out_labels = lo[:, :_MAXB, 0].astype(jnp.int32)
    return out_boxes, out_scores, out_labels


# speedup vs baseline: 3.7943x; 1.0003x over previous
import numpy as np
import jax
import jax.numpy as jnp
from jax.experimental import pallas as pl
from jax.experimental.pallas import tpu as pltpu

_NC = 80
_THR = 0.5
_MAXB = 100
_IOU = 0.1
_NEG = -1e30
_ANCH = np.array([[10., 13.], [16., 30.], [33., 23.], [30., 61.], [62., 45.],
                  [59., 119.], [116., 90.], [156., 198.], [373., 326.]],
                 dtype=np.float32)
_N = 3 * (19 * 19 + 38 * 38 + 76 * 76)
_NPAD = 22784
_SLOTS = 128
_OROWS = 104


def _build_consts():
    offx_l, offy_l, sx_l, sy_l, rw_l, rh_l = [], [], [], [], [], []
    for g, base in ((19, 6), (38, 3), (76, 0)):
        ratio = np.array([608.0 / g, 608.0 / g], dtype=np.float32)
        resc = (_ANCH[base:base + 3]
                / np.stack([ratio[1], ratio[0]]).astype(np.float32))
        resc = resc.astype(np.float32)
        shp = (g, g, 3)
        offx_l.append(np.broadcast_to(
            np.arange(g, dtype=np.float32)[None, :, None], shp).ravel())
        offy_l.append(np.broadcast_to(
            np.arange(g, dtype=np.float32)[:, None, None], shp).ravel())
        sx_l.append(np.full(shp, ratio[1], np.float32).ravel())
        sy_l.append(np.full(shp, ratio[0], np.float32).ravel())
        rw_l.append(np.broadcast_to(resc[:, 0], shp).ravel())
        rh_l.append(np.broadcast_to(resc[:, 1], shp).ravel())
    rows = []
    for lst in (offx_l, offy_l, sx_l, sy_l, rw_l, rh_l):
        v = np.concatenate(lst).astype(np.float32)
        rows.append(np.pad(v, (0, _NPAD - _N)))
    rows.append(np.zeros(_NPAD, np.float32))
    rows.append(np.zeros(_NPAD, np.float32))
    return np.stack(rows)


_CONSTS = _build_consts()


def _predict_kernel(fm_ref, cst_ref, boxes_out, scores_out, labels_out,
                    work, bxs, selsc, selix):
    offx = cst_ref[0:1, :]
    offy = cst_ref[1:2, :]
    sx = cst_ref[2:3, :]
    sy = cst_ref[3:4, :]
    rw = cst_ref[4:5, :]
    rh = cst_ref[5:6, :]
    tx = fm_ref[0:1, :]
    ty = fm_ref[1:2, :]
    tw = fm_ref[2:3, :]
    th = fm_ref[3:4, :]
    conf = fm_ref[4:5, :]
    cx = (jax.nn.sigmoid(tx) + offx) * sx
    cy = (jax.nn.sigmoid(ty) + offy) * sy
    w = (jnp.exp(tw) * rw) * sx
    h = (jnp.exp(th) * rh) * sy
    x1 = cx - w / 2
    y1 = cy - h / 2
    x2 = cx + w / 2
    y2 = cy + h / 2
    bxs[0:1, :] = x1
    bxs[1:2, :] = y1
    bxs[2:3, :] = x2
    bxs[3:4, :] = y2
    bxs[4:5, :] = (x2 - x1) * (y2 - y1)

    col = jax.lax.broadcasted_iota(jnp.int32, (1, _NPAD), 1)
    sc = jax.nn.sigmoid(fm_ref[5:5 + _NC, :]) * jax.nn.sigmoid(conf)
    work[:, :] = jnp.where((sc > _THR) & (col < _N), sc, _NEG)
    selsc[:, :] = jnp.full((_NC, _SLOTS), _NEG, jnp.float32)
    selix[:, :] = jnp.zeros((_NC, _SLOTS), jnp.float32)

    iota_l = jax.lax.broadcasted_iota(jnp.int32, (_NC, _NPAD), 1)
    slot_l = jax.lax.broadcasted_iota(jnp.int32, (_NC, _SLOTS), 1)

    boxes_out[:, :] = jnp.full((_OROWS, 128), -1.0, jnp.float32)
    scores_out[:, :] = jnp.full((_OROWS, 128), -1.0, jnp.float32)
    labels_out[:, :] = jnp.full((_OROWS, 128), -1.0, jnp.float32)

    def nms_body(carry):
        k, _ = carry
        wk = work[:, :]
        m = jnp.max(wk, axis=1, keepdims=True)
        eq = wk == m
        i = jnp.min(jnp.where(eq, iota_l, 2 ** 30), axis=1, keepdims=True)
        eq2 = iota_l == i
        X1 = bxs[0:1, :]
        Y1 = bxs[1:2, :]
        X2 = bxs[2:3, :]
        Y2 = bxs[3:4, :]
        sel_x1 = jnp.sum(jnp.where(eq2, X1, 0.0), axis=1, keepdims=True)
        sel_y1 = jnp.sum(jnp.where(eq2, Y1, 0.0), axis=1, keepdims=True)
        sel_x2 = jnp.sum(jnp.where(eq2, X2, 0.0), axis=1, keepdims=True)
        sel_y2 = jnp.sum(jnp.where(eq2, Y2, 0.0), axis=1, keepdims=True)
        a1 = (sel_x2 - sel_x1) * (sel_y2 - sel_y1)
        xx1 = jnp.maximum(sel_x1, X1)
        yy1 = jnp.maximum(sel_y1, Y1)
        xx2 = jnp.minimum(sel_x2, X2)
        yy2 = jnp.minimum(sel_y2, Y2)
        inter = jnp.maximum(xx2 - xx1, 0.0) * jnp.maximum(yy2 - yy1, 0.0)
        union = a1 + bxs[4:5, :] - inter
        iou = jnp.where(union > 0, inter / jnp.maximum(union, 1e-9), 0.0)
        work[:, :] = jnp.where((iou > _IOU) | eq2, _NEG, wk)
        onek = slot_l == k
        selsc[:, :] = jnp.where(onek, m, selsc[:, :])
        selix[:, :] = jnp.where(onek, i.astype(jnp.float32), selix[:, :])
        return k + 1, jnp.max(m) > _NEG / 2

    jax.lax.while_loop(lambda c: (c[0] < _MAXB) & c[1], nms_body, (0, True))

    keyc = jax.lax.broadcasted_iota(jnp.int32, (_NC, _SLOTS), 0)
    keys = jax.lax.broadcasted_iota(jnp.int32, (_NC, _SLOTS), 1)
    key = keyc * _SLOTS + keys
    lane = jax.lax.broadcasted_iota(jnp.int32, (1, 128), 1)
    iota_b = jax.lax.broadcasted_iota(jnp.int32, (8, _NPAD), 1)
    rowi = jax.lax.broadcasted_iota(jnp.int32, (_OROWS, 128), 0)

    def topk_body(t, _):
        s2 = selsc[:, :]
        m = jnp.max(s2)
        eq = s2 == m
        pos = jnp.min(jnp.where(eq, key, 2 ** 30))
        hit = eq & (key == pos)
        cls = jnp.sum(jnp.where(hit, keyc, 0)).astype(jnp.float32)
        nidx = jnp.sum(jnp.where(hit, selix[:, :], 0.0))
        n = nidx.astype(jnp.int32)
        selb = jnp.sum(jnp.where(iota_b == n, bxs[:, :], 0.0),
                       axis=1, keepdims=True)
        invalid = m < _NEG / 2
        row = jnp.where(lane == 0, jnp.where(invalid, -1.0, selb[0, 0]),
              jnp.where(lane == 1, jnp.where(invalid, -1.0, selb[1, 0]),
              jnp.where(lane == 2, jnp.where(invalid, -1.0, selb[2, 0]),
                        jnp.where(invalid, -1.0, selb[3, 0]))))
        hit_row = rowi == t
        boxes_out[:, :] = jnp.where(hit_row, row, boxes_out[:, :])
        scores_out[:, :] = jnp.where(
            hit_row, jnp.where(invalid, -1.0, m), scores_out[:, :])
        labels_out[:, :] = jnp.where(
            hit_row, jnp.where(invalid, -1.0, cls), labels_out[:, :])
        selsc[:, :] = jnp.where(hit, _NEG, s2)
        return 0

    jax.lax.fori_loop(0, _MAXB, topk_body, 0)


def kernel(p0, p1, p2):
    B = p0.shape[0]
    fm = jnp.concatenate(
        [p.reshape(B, -1, 5 + _NC) for p in (p0, p1, p2)], axis=1)
    fmt = jnp.transpose(fm, (0, 2, 1))
    fmt = jnp.pad(fmt, ((0, 0), (0, 3), (0, _NPAD - _N)))
    consts = jnp.asarray(_CONSTS)

    bo, so, lo = pl.pallas_call(
        _predict_kernel,
        grid=(B,),
        in_specs=[
            pl.BlockSpec((None, 88, _NPAD), lambda i: (i, 0, 0)),
            pl.BlockSpec((8, _NPAD), lambda i: (0, 0)),
        ],
        out_specs=[
            pl.BlockSpec((None, _OROWS, 128), lambda i: (i, 0, 0)),
            pl.BlockSpec((None, _OROWS, 128), lambda i: (i, 0, 0)),
            pl.BlockSpec((None, _OROWS, 128), lambda i: (i, 0, 0)),
        ],
        out_shape=[
            jax.ShapeDtypeStruct((B, _OROWS, 128), jnp.float32),
            jax.ShapeDtypeStruct((B, _OROWS, 128), jnp.float32),
            jax.ShapeDtypeStruct((B, _OROWS, 128), jnp.float32),
        ],
        scratch_shapes=[
            pltpu.VMEM((_NC, _NPAD), jnp.float32),
            pltpu.VMEM((8, _NPAD), jnp.float32),
            pltpu.VMEM((_NC, _SLOTS), jnp.float32),
            pltpu.VMEM((_NC, _SLOTS), jnp.float32),
        ],
        compiler_params=pltpu.CompilerParams(
            dimension_semantics=("parallel",)),
    )(fmt, consts)

    out_boxes = bo[:, :_MAXB, :4]
    out_scores = so[:, :_MAXB, 0]
    out_labels = lo[:, :_MAXB, 0].astype(jnp.int32)
    return out_boxes, out_scores, out_labels


# speedup vs baseline: 3.9903x; 1.0517x over previous
import numpy as np
import jax
import jax.numpy as jnp
from jax.experimental import pallas as pl
from jax.experimental.pallas import tpu as pltpu

_NC = 80
_THR = 0.5
_MAXB = 100
_IOU = 0.1
_NEG = -1e30
_ANCH = np.array([[10., 13.], [16., 30.], [33., 23.], [30., 61.], [62., 45.],
                  [59., 119.], [116., 90.], [156., 198.], [373., 326.]],
                 dtype=np.float32)
_N = 3 * (19 * 19 + 38 * 38 + 76 * 76)
_NPAD = 22784
_SLOTS = 128
_OROWS = 104


def _build_consts():
    offx_l, offy_l, sx_l, sy_l, rw_l, rh_l = [], [], [], [], [], []
    for g, base in ((19, 6), (38, 3), (76, 0)):
        ratio = np.array([608.0 / g, 608.0 / g], dtype=np.float32)
        resc = (_ANCH[base:base + 3]
                / np.stack([ratio[1], ratio[0]]).astype(np.float32))
        resc = resc.astype(np.float32)
        shp = (g, g, 3)
        offx_l.append(np.broadcast_to(
            np.arange(g, dtype=np.float32)[None, :, None], shp).ravel())
        offy_l.append(np.broadcast_to(
            np.arange(g, dtype=np.float32)[:, None, None], shp).ravel())
        sx_l.append(np.full(shp, ratio[1], np.float32).ravel())
        sy_l.append(np.full(shp, ratio[0], np.float32).ravel())
        rw_l.append(np.broadcast_to(resc[:, 0], shp).ravel())
        rh_l.append(np.broadcast_to(resc[:, 1], shp).ravel())
    rows = []
    for lst in (offx_l, offy_l, sx_l, sy_l, rw_l, rh_l):
        v = np.concatenate(lst).astype(np.float32)
        rows.append(np.pad(v, (0, _NPAD - _N)))
    rows.append(np.zeros(_NPAD, np.float32))
    rows.append(np.zeros(_NPAD, np.float32))
    return np.stack(rows)


_CONSTS = _build_consts()


def _predict_kernel(fm_ref, cst_ref, boxes_out, scores_out, labels_out,
                    work, bxs, selsc, selix):
    offx = cst_ref[0:1, :]
    offy = cst_ref[1:2, :]
    sx = cst_ref[2:3, :]
    sy = cst_ref[3:4, :]
    rw = cst_ref[4:5, :]
    rh = cst_ref[5:6, :]
    tx = fm_ref[0:1, :]
    ty = fm_ref[1:2, :]
    tw = fm_ref[2:3, :]
    th = fm_ref[3:4, :]
    conf = fm_ref[4:5, :]
    cx = (jax.nn.sigmoid(tx) + offx) * sx
    cy = (jax.nn.sigmoid(ty) + offy) * sy
    w = (jnp.exp(tw) * rw) * sx
    h = (jnp.exp(th) * rh) * sy
    x1 = cx - w / 2
    y1 = cy - h / 2
    x2 = cx + w / 2
    y2 = cy + h / 2
    bxs[0:1, :] = x1
    bxs[1:2, :] = y1
    bxs[2:3, :] = x2
    bxs[3:4, :] = y2
    bxs[4:5, :] = (x2 - x1) * (y2 - y1)

    col = jax.lax.broadcasted_iota(jnp.int32, (1, _NPAD), 1)
    sc = jax.nn.sigmoid(fm_ref[5:5 + _NC, :]) * jax.nn.sigmoid(conf)
    work[:, :] = jnp.where((sc > _THR) & (col < _N), sc, _NEG)
    selsc[:, :] = jnp.full((_NC, _SLOTS), _NEG, jnp.float32)
    selix[:, :] = jnp.zeros((_NC, _SLOTS), jnp.float32)

    iota_l = jax.lax.broadcasted_iota(jnp.int32, (_NC, _NPAD), 1)
    slot_l = jax.lax.broadcasted_iota(jnp.int32, (_NC, _SLOTS), 1)

    boxes_out[:, :] = jnp.full((_OROWS, 128), -1.0, jnp.float32)
    scores_out[:, :] = jnp.full((_OROWS, 128), -1.0, jnp.float32)
    labels_out[:, :] = jnp.full((_OROWS, 128), -1.0, jnp.float32)

    def nms_body(carry):
        k, m = carry
        wk = work[:, :]
        i = jnp.min(jnp.where(wk == m, iota_l, 2 ** 30),
                    axis=1, keepdims=True)
        eq2 = iota_l == i
        X1 = bxs[0:1, :]
        Y1 = bxs[1:2, :]
        X2 = bxs[2:3, :]
        Y2 = bxs[3:4, :]
        sel_x1 = jnp.sum(jnp.where(eq2, X1, 0.0), axis=1, keepdims=True)
        sel_y1 = jnp.sum(jnp.where(eq2, Y1, 0.0), axis=1, keepdims=True)
        sel_x2 = jnp.sum(jnp.where(eq2, X2, 0.0), axis=1, keepdims=True)
        sel_y2 = jnp.sum(jnp.where(eq2, Y2, 0.0), axis=1, keepdims=True)
        a1 = (sel_x2 - sel_x1) * (sel_y2 - sel_y1)
        xx1 = jnp.maximum(sel_x1, X1)
        yy1 = jnp.maximum(sel_y1, Y1)
        xx2 = jnp.minimum(sel_x2, X2)
        yy2 = jnp.minimum(sel_y2, Y2)
        inter = jnp.maximum(xx2 - xx1, 0.0) * jnp.maximum(yy2 - yy1, 0.0)
        union = a1 + bxs[4:5, :] - inter
        iou = jnp.where(union > 0, inter / jnp.maximum(union, 1e-9), 0.0)
        new_work = jnp.where(iou > _IOU, _NEG, wk)
        work[:, :] = new_work
        m_next = jnp.max(new_work, axis=1, keepdims=True)
        onek = slot_l == k
        selsc[:, :] = jnp.where(onek, m, selsc[:, :])
        selix[:, :] = jnp.where(onek, i.astype(jnp.float32), selix[:, :])
        return k + 1, m_next

    m0 = jnp.max(work[:, :], axis=1, keepdims=True)
    jax.lax.while_loop(
        lambda c: (c[0] < _MAXB) & (jnp.max(c[1]) > _NEG / 2),
        nms_body, (0, m0))

    keyc = jax.lax.broadcasted_iota(jnp.int32, (_NC, _SLOTS), 0)
    keys = jax.lax.broadcasted_iota(jnp.int32, (_NC, _SLOTS), 1)
    key = keyc * _SLOTS + keys
    lane = jax.lax.broadcasted_iota(jnp.int32, (1, 128), 1)
    iota_b = jax.lax.broadcasted_iota(jnp.int32, (8, _NPAD), 1)
    rowi = jax.lax.broadcasted_iota(jnp.int32, (_OROWS, 128), 0)

    def topk_body(t, _):
        s2 = selsc[:, :]
        m = jnp.max(s2)
        eq = s2 == m
        pos = jnp.min(jnp.where(eq, key, 2 ** 30))
        hit = eq & (key == pos)
        cls = jnp.sum(jnp.where(hit, keyc, 0)).astype(jnp.float32)
        nidx = jnp.sum(jnp.where(hit, selix[:, :], 0.0))
        n = nidx.astype(jnp.int32)
        selb = jnp.sum(jnp.where(iota_b == n, bxs[:, :], 0.0),
                       axis=1, keepdims=True)
        invalid = m < _NEG / 2
        row = jnp.where(lane == 0, jnp.where(invalid, -1.0, selb[0, 0]),
              jnp.where(lane == 1, jnp.where(invalid, -1.0, selb[1, 0]),
              jnp.where(lane == 2, jnp.where(invalid, -1.0, selb[2, 0]),
                        jnp.where(invalid, -1.0, selb[3, 0]))))
        hit_row = rowi == t
        boxes_out[:, :] = jnp.where(hit_row, row, boxes_out[:, :])
        scores_out[:, :] = jnp.where(
            hit_row, jnp.where(invalid, -1.0, m), scores_out[:, :])
        labels_out[:, :] = jnp.where(
            hit_row, jnp.where(invalid, -1.0, cls), labels_out[:, :])
        selsc[:, :] = jnp.where(hit, _NEG, s2)
        return 0

    jax.lax.fori_loop(0, _MAXB, topk_body, 0)


def kernel(p0, p1, p2):
    B = p0.shape[0]
    fm = jnp.concatenate(
        [p.reshape(B, -1, 5 + _NC) for p in (p0, p1, p2)], axis=1)
    fmt = jnp.transpose(fm, (0, 2, 1))
    fmt = jnp.pad(fmt, ((0, 0), (0, 3), (0, _NPAD - _N)))
    consts = jnp.asarray(_CONSTS)

    bo, so, lo = pl.pallas_call(
        _predict_kernel,
        grid=(B,),
        in_specs=[
            pl.BlockSpec((None, 88, _NPAD), lambda i: (i, 0, 0)),
            pl.BlockSpec((8, _NPAD), lambda i: (0, 0)),
        ],
        out_specs=[
            pl.BlockSpec((None, _OROWS, 128), lambda i: (i, 0, 0)),
            pl.BlockSpec((None, _OROWS, 128), lambda i: (i, 0, 0)),
            pl.BlockSpec((None, _OROWS, 128), lambda i: (i, 0, 0)),
        ],
        out_shape=[
            jax.ShapeDtypeStruct((B, _OROWS, 128), jnp.float32),
            jax.ShapeDtypeStruct((B, _OROWS, 128), jnp.float32),
            jax.ShapeDtypeStruct((B, _OROWS, 128), jnp.float32),
        ],
        scratch_shapes=[
            pltpu.VMEM((_NC, _NPAD), jnp.float32),
            pltpu.VMEM((8, _NPAD), jnp.float32),
            pltpu.VMEM((_NC, _SLOTS), jnp.float32),
            pltpu.VMEM((_NC, _SLOTS), jnp.float32),
        ],
        compiler_params=pltpu.CompilerParams(
            dimension_semantics=("parallel",)),
    )(fmt, consts)

    out_boxes = bo[:, :_MAXB, :4]
    out_scores = so[:, :_MAXB, 0]
    out_labels = lo[:, :_MAXB, 0].astype(jnp.int32)
    return out_boxes, out_scores, out_labels
